# Initial kernel scaffold; baseline (speedup 1.0000x reference)
import functools

import jax
import jax.numpy as jnp
from jax import lax
from jax.experimental import pallas as pl
from jax.experimental.pallas import tpu as pltpu
from jax.experimental.pallas import tpu_sc as plsc

N_NODES = 10000
N_EDGES = 320000
D = 128

NC = 2
NS = 16
NW = NC * NS
E_PER_W = N_EDGES // NW
K = 80
CHUNKS = E_PER_W // K
ROWS_PER_TILE = N_NODES // NS
ZROWS = 125
DEGW = 16


def _sc_agg_body(with_deg, *refs):
    if with_deg:
        (x_hbm, src_hbm, dst_hbm, acc_out, deg_out,
         srcv, dstv, rows, zbuf, onesv, zdeg, acc_sh, deg_sh, sem) = refs
    else:
        (x_hbm, src_hbm, dst_hbm, acc_out,
         srcv, dstv, rows, zbuf, acc_sh, sem) = refs

    cid = lax.axis_index("c")
    sid = lax.axis_index("s")
    wid = sid * NC + cid
    base = wid * E_PER_W

    zero16 = jnp.zeros((16,), jnp.float32)

    def zrow(i, _):
        for j in range(D // 16):
            zbuf[i, pl.ds(j * 16, 16)] = zero16
        return 0
    lax.fori_loop(0, ZROWS, zrow, 0)
    r0 = sid * ROWS_PER_TILE
    for r in range(ROWS_PER_TILE // ZROWS):
        pltpu.sync_copy(zbuf, acc_sh.at[pl.ds(r0 + r * ZROWS, ZROWS)])

    if with_deg:
        one16 = jnp.ones((16,), jnp.float32)

        def orow(i, _):
            onesv[i, :] = one16
            zdeg[i, :] = zero16
            return 0
        lax.fori_loop(0, ZROWS, orow, 0)
        for r in range(ROWS_PER_TILE // ZROWS):
            pltpu.sync_copy(zdeg.at[pl.ds(0, ZROWS)],
                            deg_sh.at[pl.ds(r0 + r * ZROWS, ZROWS)])

    plsc.subcore_barrier()

    def body(i, _):
        off = base + i * K
        pltpu.sync_copy(src_hbm.at[pl.ds(off, K)], srcv)
        pltpu.sync_copy(dst_hbm.at[pl.ds(off, K)], dstv)
        pltpu.async_copy(x_hbm.at[srcv], rows, sem).wait()
        pltpu.sync_copy(rows, acc_sh.at[dstv], add=True)
        if with_deg:
            pltpu.sync_copy(onesv.at[pl.ds(0, K)], deg_sh.at[dstv], add=True)
        return 0
    lax.fori_loop(0, CHUNKS, body, 0)

    plsc.subcore_barrier()

    for r in range(ROWS_PER_TILE // ZROWS):
        rr = r0 + r * ZROWS
        pltpu.sync_copy(acc_sh.at[pl.ds(rr, ZROWS)], zbuf)
        pltpu.sync_copy(zbuf, acc_out.at[cid, pl.ds(rr, ZROWS)])
    if with_deg:
        for r in range(ROWS_PER_TILE // ZROWS):
            rr = r0 + r * ZROWS
            pltpu.sync_copy(deg_sh.at[pl.ds(rr, ZROWS)], zdeg)
            pltpu.sync_copy(zdeg, deg_out.at[cid, pl.ds(rr, ZROWS)])


def _make_sc_agg(with_deg):
    mesh = plsc.VectorSubcoreMesh(core_axis_name="c", subcore_axis_name="s")
    out_type = [jax.ShapeDtypeStruct((NC, N_NODES, D), jnp.float32)]
    scratch = [
        pltpu.VMEM((K,), jnp.int32),
        pltpu.VMEM((K,), jnp.int32),
        pltpu.VMEM((K, D), jnp.float32),
        pltpu.VMEM((ZROWS, D), jnp.float32),
    ]
    if with_deg:
        out_type.append(jax.ShapeDtypeStruct((NC, N_NODES, DEGW), jnp.float32))
        scratch += [
            pltpu.VMEM((ZROWS, DEGW), jnp.float32),
            pltpu.VMEM((ZROWS, DEGW), jnp.float32),
        ]
    scratch.append(pltpu.VMEM_SHARED((N_NODES, D), jnp.float32))
    if with_deg:
        scratch.append(pltpu.VMEM_SHARED((N_NODES, DEGW), jnp.float32))
    scratch.append(pltpu.SemaphoreType.DMA)

    return pl.kernel(
        functools.partial(_sc_agg_body, with_deg),
        out_type=out_type,
        mesh=mesh,
        scratch_types=scratch,
    )


_sc_agg_deg = _make_sc_agg(True)
_sc_agg = _make_sc_agg(False)


R_BLK = 2000


def _tc_dense_body(prelu, final, *refs):
    if final:
        (acc_ref, deg_ref, h_ref, wl_ref, bl_ref, wr_ref, a_ref,
         wp_ref, bp_ref, out_ref) = refs
    else:
        (acc_ref, deg_ref, h_ref, wl_ref, bl_ref, wr_ref, a_ref,
         out_ref) = refs
    acc = acc_ref[0] + acc_ref[1]
    deg = deg_ref[0, :, 0] + deg_ref[1, :, 0]
    mean = acc * (1.0 / jnp.clip(deg, 1.0, None))[:, None]
    h = h_ref[...]
    out = (jnp.dot(mean, wl_ref[...], preferred_element_type=jnp.float32)
           + bl_ref[...][None, :]
           + jnp.dot(h, wr_ref[...], preferred_element_type=jnp.float32))
    if prelu:
        a = a_ref[0, 0]
        out = jnp.where(out >= 0, out, a * out)
    if final:
        lvl = jnp.dot(out, wp_ref[...], preferred_element_type=jnp.float32)
        out_ref[...] = lvl + bp_ref[...][None, :]
    else:
        out_ref[...] = out


def _make_tc_dense(prelu, final):
    n_blk = N_NODES // R_BLK
    full = lambda i: (0, 0)
    in_specs = [
        pl.BlockSpec((NC, R_BLK, D), lambda i: (0, i, 0)),
        pl.BlockSpec((NC, R_BLK, DEGW), lambda i: (0, i, 0)),
        pl.BlockSpec((R_BLK, D), lambda i: (i, 0)),
        pl.BlockSpec((D, D), full),
        pl.BlockSpec((D,), lambda i: (0,)),
        pl.BlockSpec((D, D), full),
        pl.BlockSpec((1, 1), full),
    ]
    if final:
        in_specs += [
            pl.BlockSpec((D, 1), full),
            pl.BlockSpec((1,), lambda i: (0,)),
        ]
        out_spec = pl.BlockSpec((R_BLK, 1), lambda i: (i, 0))
        out_shape = jax.ShapeDtypeStruct((N_NODES, 1), jnp.float32)
    else:
        out_spec = pl.BlockSpec((R_BLK, D), lambda i: (i, 0))
        out_shape = jax.ShapeDtypeStruct((N_NODES, D), jnp.float32)
    return pl.pallas_call(
        functools.partial(_tc_dense_body, prelu, final),
        grid=(n_blk,),
        in_specs=in_specs,
        out_specs=out_spec,
        out_shape=out_shape,
    )


_tc_mid = _make_tc_dense(True, False)
_tc_last = _make_tc_dense(False, True)


def kernel(x, edge_index, Wl1, bl1, Wr1, Wl2, bl2, Wr2, Wl3, bl3, Wr3,
           a, Wp, bp):
    src = edge_index[0].astype(jnp.int32)
    dst = edge_index[1].astype(jnp.int32)
    a2 = jnp.asarray(a, jnp.float32).reshape(1, 1)

    acc1, degp = _sc_agg_deg(x, src, dst)
    h1 = _tc_mid(acc1, degp, x, Wl1, bl1, Wr1, a2)
    acc2 = _sc_agg(h1, src, dst)
    h2 = _tc_mid(acc2, degp, h1, Wl2, bl2, Wr2, a2)
    acc3 = _sc_agg(h2, src, dst)
    out = _tc_last(acc3, degp, h2, Wl3, bl3, Wr3, a2, Wp, bp)
    return out[:, 0]


# speedup vs baseline: 4.8362x; 4.8362x over previous
import functools

import jax
import jax.numpy as jnp
from jax import lax
from jax.experimental import pallas as pl
from jax.experimental.pallas import tpu as pltpu
from jax.experimental.pallas import tpu_sc as plsc

N_NODES = 10000
N_EDGES = 320000
D = 128

NC = 2
NS = 16
NW = NC * NS
E_PER_W = N_EDGES // NW
K = 80
CHUNKS = E_PER_W // K
N_PAD = 10240
ROWS_PER_TILE = N_PAD // NS
ZROWS = 128
DEGW = 128


def _sc_agg_body(x_hbm, src_hbm, dst_hbm, acc_out,
                 srcv, dstv, rows, zbuf, acc_sh, sem):
    cid = lax.axis_index("c")
    sid = lax.axis_index("s")
    wid = sid * NC + cid
    base = wid * E_PER_W

    zero16 = jnp.zeros((16,), jnp.float32)

    def zrow(i, _):
        for j in range(D // 16):
            zbuf[i, pl.ds(j * 16, 16)] = zero16
        return 0
    lax.fori_loop(0, ZROWS, zrow, 0)
    r0 = sid * ROWS_PER_TILE
    for r in range(ROWS_PER_TILE // ZROWS):
        pltpu.sync_copy(zbuf, acc_sh.at[pl.ds(r0 + r * ZROWS, ZROWS)])

    plsc.subcore_barrier()

    def body(i, _):
        off = base + i * K
        pltpu.sync_copy(src_hbm.at[pl.ds(off, K)], srcv)
        pltpu.sync_copy(dst_hbm.at[pl.ds(off, K)], dstv)
        pltpu.async_copy(x_hbm.at[srcv], rows, sem).wait()
        pltpu.sync_copy(rows, acc_sh.at[dstv], add=True)
        return 0
    lax.fori_loop(0, CHUNKS, body, 0)

    plsc.subcore_barrier()

    for r in range(ROWS_PER_TILE // ZROWS):
        rr = r0 + r * ZROWS
        pltpu.sync_copy(acc_sh.at[pl.ds(rr, ZROWS)], zbuf)
        pltpu.sync_copy(zbuf, acc_out.at[cid, pl.ds(rr, ZROWS)])


def _make_sc_agg():
    mesh = plsc.VectorSubcoreMesh(core_axis_name="c", subcore_axis_name="s",
                                  num_cores=NC, num_subcores=NS)
    return pl.kernel(
        _sc_agg_body,
        out_type=[jax.ShapeDtypeStruct((NC, N_PAD, D), jnp.float32)],
        mesh=mesh,
        scratch_types=[
            pltpu.VMEM((K,), jnp.int32),
            pltpu.VMEM((K,), jnp.int32),
            pltpu.VMEM((K, D), jnp.float32),
            pltpu.VMEM((ZROWS, D), jnp.float32),
            pltpu.VMEM_SHARED((N_PAD, D), jnp.float32),
            pltpu.SemaphoreType.DMA,
        ],
    )


def _sc_deg_body(dst_hbm, deg_out, dstv, onesv, zdeg, deg_sh):
    cid = lax.axis_index("c")
    sid = lax.axis_index("s")
    wid = sid * NC + cid
    base = wid * E_PER_W

    zero16 = jnp.zeros((16,), jnp.float32)
    one16 = jnp.ones((16,), jnp.float32)

    def orow(i, _):
        for j in range(DEGW // 16):
            onesv[i, pl.ds(j * 16, 16)] = one16
            zdeg[i, pl.ds(j * 16, 16)] = zero16
        return 0
    lax.fori_loop(0, ZROWS, orow, 0)
    r0 = sid * ROWS_PER_TILE
    for r in range(ROWS_PER_TILE // ZROWS):
        pltpu.sync_copy(zdeg, deg_sh.at[pl.ds(r0 + r * ZROWS, ZROWS)])

    plsc.subcore_barrier()

    def body(i, _):
        off = base + i * K
        pltpu.sync_copy(dst_hbm.at[pl.ds(off, K)], dstv)
        pltpu.sync_copy(onesv.at[pl.ds(0, K)], deg_sh.at[dstv], add=True)
        return 0
    lax.fori_loop(0, CHUNKS, body, 0)

    plsc.subcore_barrier()

    for r in range(ROWS_PER_TILE // ZROWS):
        rr = r0 + r * ZROWS
        pltpu.sync_copy(deg_sh.at[pl.ds(rr, ZROWS)], zdeg)
        pltpu.sync_copy(zdeg, deg_out.at[cid, pl.ds(rr, ZROWS)])


def _make_sc_deg():
    mesh = plsc.VectorSubcoreMesh(core_axis_name="c", subcore_axis_name="s",
                                  num_cores=NC, num_subcores=NS)
    return pl.kernel(
        _sc_deg_body,
        out_type=[jax.ShapeDtypeStruct((NC, N_PAD, DEGW), jnp.float32)],
        mesh=mesh,
        scratch_types=[
            pltpu.VMEM((K,), jnp.int32),
            pltpu.VMEM((ZROWS, DEGW), jnp.float32),
            pltpu.VMEM((ZROWS, DEGW), jnp.float32),
            pltpu.VMEM_SHARED((N_PAD, DEGW), jnp.float32),
        ],
    )


_sc_agg = _make_sc_agg()
_sc_deg = _make_sc_deg()


R_BLK = 2000


def _tc_dense_body(prelu, final, *refs):
    if final:
        (acc_ref, deg_ref, h_ref, wl_ref, bl_ref, wr_ref, a_ref,
         wp_ref, bp_ref, out_ref) = refs
    else:
        (acc_ref, deg_ref, h_ref, wl_ref, bl_ref, wr_ref, a_ref,
         out_ref) = refs
    acc = acc_ref[0] + acc_ref[1]
    deg = deg_ref[0, :, 0] + deg_ref[1, :, 0]
    mean = acc * (1.0 / jnp.clip(deg, 1.0, None))[:, None]
    h = h_ref[...]
    out = (jnp.dot(mean, wl_ref[...], preferred_element_type=jnp.float32)
           + bl_ref[...][None, :]
           + jnp.dot(h, wr_ref[...], preferred_element_type=jnp.float32))
    if prelu:
        a = a_ref[0, 0]
        out = jnp.where(out >= 0, out, a * out)
    if final:
        lvl = jnp.dot(out, wp_ref[...], preferred_element_type=jnp.float32)
        out_ref[...] = lvl + bp_ref[...][None, :]
    else:
        out_ref[...] = out


def _make_tc_dense(prelu, final):
    n_blk = N_NODES // R_BLK
    full = lambda i: (0, 0)
    in_specs = [
        pl.BlockSpec((NC, R_BLK, D), lambda i: (0, i, 0)),
        pl.BlockSpec((NC, R_BLK, DEGW), lambda i: (0, i, 0)),
        pl.BlockSpec((R_BLK, D), lambda i: (i, 0)),
        pl.BlockSpec((D, D), full),
        pl.BlockSpec((D,), lambda i: (0,)),
        pl.BlockSpec((D, D), full),
        pl.BlockSpec((1, 1), full),
    ]
    if final:
        in_specs += [
            pl.BlockSpec((D, 1), full),
            pl.BlockSpec((1,), lambda i: (0,)),
        ]
        out_spec = pl.BlockSpec((R_BLK, 1), lambda i: (i, 0))
        out_shape = jax.ShapeDtypeStruct((N_NODES, 1), jnp.float32)
    else:
        out_spec = pl.BlockSpec((R_BLK, D), lambda i: (i, 0))
        out_shape = jax.ShapeDtypeStruct((N_NODES, D), jnp.float32)
    return pl.pallas_call(
        functools.partial(_tc_dense_body, prelu, final),
        grid=(n_blk,),
        in_specs=in_specs,
        out_specs=out_spec,
        out_shape=out_shape,
    )


_tc_mid = _make_tc_dense(True, False)
_tc_last = _make_tc_dense(False, True)


def kernel(x, edge_index, Wl1, bl1, Wr1, Wl2, bl2, Wr2, Wl3, bl3, Wr3,
           a, Wp, bp):
    src = edge_index[0].astype(jnp.int32)
    dst = edge_index[1].astype(jnp.int32)
    a2 = jnp.asarray(a, jnp.float32).reshape(1, 1)

    degp, = _sc_deg(dst)
    acc1, = _sc_agg(x, src, dst)
    h1 = _tc_mid(acc1, degp, x, Wl1, bl1, Wr1, a2)
    acc2, = _sc_agg(h1, src, dst)
    h2 = _tc_mid(acc2, degp, h1, Wl2, bl2, Wr2, a2)
    acc3, = _sc_agg(h2, src, dst)
    out = _tc_last(acc3, degp, h2, Wl3, bl3, Wr3, a2, Wp, bp)
    return out[:, 0]


# speedup vs baseline: 9.2270x; 1.9079x over previous
import functools

import jax
import jax.numpy as jnp
from jax import lax
from jax.experimental import pallas as pl
from jax.experimental.pallas import tpu as pltpu
from jax.experimental.pallas import tpu_sc as plsc

N_NODES = 10000
N_EDGES = 320000
D = 128

NC = 2
NS = 16
NW = NC * NS
E_PER_W = N_EDGES // NW
K = 50
G = 8
NG = E_PER_W // (G * K)
N_PAD = 10240
ROWS_PER_TILE = N_PAD // NS
ZROWS = 64
DEGW = 128


def _sc_agg_body(x_hbm, srcG_hbm, dstG_hbm, acc_out,
                 srcT, dstT, zbuf, acc_sh,
                 rows0, rows1, g0, g1, s0, s1, tsem):
    rows = (rows0, rows1)
    gsem = (g0, g1)
    ssem = (s0, s1)

    cid = lax.axis_index("c")
    sid = lax.axis_index("s")
    wid = sid * NC + cid

    zero16 = jnp.zeros((16,), jnp.float32)

    def zrow(i, _):
        for j in range(D // 16):
            zbuf[i, pl.ds(j * 16, 16)] = zero16
        return 0
    lax.fori_loop(0, ZROWS, zrow, 0)
    r0 = sid * ROWS_PER_TILE
    for r in range(ROWS_PER_TILE // ZROWS):
        pltpu.sync_copy(zbuf, acc_sh.at[pl.ds(r0 + r * ZROWS, ZROWS)])

    plsc.subcore_barrier()

    pltpu.sync_copy(srcG_hbm.at[wid, 0], srcT.at[0])
    pltpu.sync_copy(dstG_hbm.at[wid, 0], dstT.at[0])
    pltpu.async_copy(srcG_hbm.at[wid, 1], srcT.at[1], tsem)
    pltpu.async_copy(dstG_hbm.at[wid, 1], dstT.at[1], tsem)
    pltpu.async_copy(x_hbm.at[srcT.at[0, 0]], rows[0], gsem[0])
    pltpu.async_copy(x_hbm.at[srcT.at[0, 1]], rows[1], gsem[1])

    def outer(grp, _):
        h = lax.rem(grp, 2)
        last = grp >= NG - 1
        for j in range(G):
            b = j % 2
            pltpu.make_async_copy(
                x_hbm.at[srcT.at[h, j]], rows[b], gsem[b]).wait()
            pltpu.async_copy(rows[b], acc_sh.at[dstT.at[h, j]], ssem[b],
                             add=True)
            if j == 6:
                @pl.when(jnp.logical_not(last))
                def _():
                    pltpu.make_async_copy(
                        srcG_hbm.at[wid, 0], srcT.at[0], tsem).wait()
                    pltpu.make_async_copy(
                        dstG_hbm.at[wid, 0], dstT.at[0], tsem).wait()
            pltpu.make_async_copy(
                rows[b], acc_sh.at[dstT.at[h, j]], ssem[b]).wait()
            if j < G - 2:
                pltpu.async_copy(x_hbm.at[srcT.at[h, j + 2]], rows[b],
                                 gsem[b])
            else:
                @pl.when(jnp.logical_not(last))
                def _():
                    pltpu.async_copy(
                        x_hbm.at[srcT.at[1 - h, j - (G - 2)]], rows[b],
                        gsem[b])

        @pl.when(grp + 2 < NG)
        def _():
            pltpu.async_copy(srcG_hbm.at[wid, grp + 2], srcT.at[h], tsem)
            pltpu.async_copy(dstG_hbm.at[wid, grp + 2], dstT.at[h], tsem)
        return 0
    lax.fori_loop(0, NG, outer, 0)

    plsc.subcore_barrier()

    for r in range(ROWS_PER_TILE // ZROWS):
        rr = r0 + r * ZROWS
        pltpu.sync_copy(acc_sh.at[pl.ds(rr, ZROWS)], zbuf)
        pltpu.sync_copy(zbuf, acc_out.at[cid, pl.ds(rr, ZROWS)])


def _make_sc_agg():
    mesh = plsc.VectorSubcoreMesh(core_axis_name="c", subcore_axis_name="s",
                                  num_cores=NC, num_subcores=NS)
    return pl.kernel(
        _sc_agg_body,
        out_type=[jax.ShapeDtypeStruct((NC, N_PAD, D), jnp.float32)],
        mesh=mesh,
        scratch_types=[
            pltpu.VMEM((2, G, K), jnp.int32),
            pltpu.VMEM((2, G, K), jnp.int32),
            pltpu.VMEM((ZROWS, D), jnp.float32),
            pltpu.VMEM_SHARED((N_PAD, D), jnp.float32),
            pltpu.VMEM((K, D), jnp.float32),
            pltpu.VMEM((K, D), jnp.float32),
            pltpu.SemaphoreType.DMA,
            pltpu.SemaphoreType.DMA,
            pltpu.SemaphoreType.DMA,
            pltpu.SemaphoreType.DMA,
            pltpu.SemaphoreType.DMA,
        ],
    )


def _sc_deg_body(dstG_hbm, deg_out, dstT, onesv, zdeg, deg_sh, sem):
    cid = lax.axis_index("c")
    sid = lax.axis_index("s")
    wid = sid * NC + cid

    zero16 = jnp.zeros((16,), jnp.float32)
    one16 = jnp.ones((16,), jnp.float32)

    def orow(i, _):
        for j in range(DEGW // 16):
            onesv[i, pl.ds(j * 16, 16)] = one16
            zdeg[i, pl.ds(j * 16, 16)] = zero16
        return 0
    lax.fori_loop(0, ZROWS, orow, 0)
    r0 = sid * ROWS_PER_TILE
    for r in range(ROWS_PER_TILE // ZROWS):
        pltpu.sync_copy(zdeg, deg_sh.at[pl.ds(r0 + r * ZROWS, ZROWS)])

    plsc.subcore_barrier()

    ones_rows = onesv.at[pl.ds(0, K)]

    def grp_body(grp, _):
        pltpu.sync_copy(dstG_hbm.at[wid, grp], dstT)
        for j in range(G):
            pltpu.async_copy(ones_rows, deg_sh.at[dstT.at[j]], sem, add=True)
        for j in range(G):
            pltpu.make_async_copy(ones_rows, deg_sh.at[dstT.at[0]],
                                  sem).wait()
        return 0
    lax.fori_loop(0, NG, grp_body, 0)

    plsc.subcore_barrier()

    for r in range(ROWS_PER_TILE // ZROWS):
        rr = r0 + r * ZROWS
        pltpu.sync_copy(deg_sh.at[pl.ds(rr, ZROWS)], zdeg)
        pltpu.sync_copy(zdeg, deg_out.at[cid, pl.ds(rr, ZROWS)])


def _make_sc_deg():
    mesh = plsc.VectorSubcoreMesh(core_axis_name="c", subcore_axis_name="s",
                                  num_cores=NC, num_subcores=NS)
    return pl.kernel(
        _sc_deg_body,
        out_type=[jax.ShapeDtypeStruct((NC, N_PAD, DEGW), jnp.float32)],
        mesh=mesh,
        scratch_types=[
            pltpu.VMEM((G, K), jnp.int32),
            pltpu.VMEM((ZROWS, DEGW), jnp.float32),
            pltpu.VMEM((ZROWS, DEGW), jnp.float32),
            pltpu.VMEM_SHARED((N_PAD, DEGW), jnp.float32),
            pltpu.SemaphoreType.DMA,
        ],
    )


_sc_agg = _make_sc_agg()
_sc_deg = _make_sc_deg()


R_BLK = 2000


def _tc_dense_body(prelu, final, *refs):
    if final:
        (acc_ref, deg_ref, h_ref, wl_ref, bl_ref, wr_ref, a_ref,
         wp_ref, bp_ref, out_ref) = refs
    else:
        (acc_ref, deg_ref, h_ref, wl_ref, bl_ref, wr_ref, a_ref,
         out_ref) = refs
    acc = acc_ref[0] + acc_ref[1]
    deg = deg_ref[0, :, 0] + deg_ref[1, :, 0]
    mean = acc * (1.0 / jnp.clip(deg, 1.0, None))[:, None]
    h = h_ref[...]
    out = (jnp.dot(mean, wl_ref[...], preferred_element_type=jnp.float32)
           + bl_ref[...][None, :]
           + jnp.dot(h, wr_ref[...], preferred_element_type=jnp.float32))
    if prelu:
        a = a_ref[0, 0]
        out = jnp.where(out >= 0, out, a * out)
    if final:
        lvl = jnp.dot(out, wp_ref[...], preferred_element_type=jnp.float32)
        out_ref[...] = lvl + bp_ref[...][None, :]
    else:
        out_ref[...] = out


def _make_tc_dense(prelu, final):
    n_blk = N_NODES // R_BLK
    full = lambda i: (0, 0)
    in_specs = [
        pl.BlockSpec((NC, R_BLK, D), lambda i: (0, i, 0)),
        pl.BlockSpec((NC, R_BLK, DEGW), lambda i: (0, i, 0)),
        pl.BlockSpec((R_BLK, D), lambda i: (i, 0)),
        pl.BlockSpec((D, D), full),
        pl.BlockSpec((D,), lambda i: (0,)),
        pl.BlockSpec((D, D), full),
        pl.BlockSpec((1, 1), full),
    ]
    if final:
        in_specs += [
            pl.BlockSpec((D, 1), full),
            pl.BlockSpec((1,), lambda i: (0,)),
        ]
        out_spec = pl.BlockSpec((R_BLK, 1), lambda i: (i, 0))
        out_shape = jax.ShapeDtypeStruct((N_NODES, 1), jnp.float32)
    else:
        out_spec = pl.BlockSpec((R_BLK, D), lambda i: (i, 0))
        out_shape = jax.ShapeDtypeStruct((N_NODES, D), jnp.float32)
    return pl.pallas_call(
        functools.partial(_tc_dense_body, prelu, final),
        grid=(n_blk,),
        in_specs=in_specs,
        out_specs=out_spec,
        out_shape=out_shape,
    )


_tc_mid = _make_tc_dense(True, False)
_tc_last = _make_tc_dense(False, True)


def kernel(x, edge_index, Wl1, bl1, Wr1, Wl2, bl2, Wr2, Wl3, bl3, Wr3,
           a, Wp, bp):
    srcG = edge_index[0].astype(jnp.int32).reshape(NW, NG, G, K)
    dstG = edge_index[1].astype(jnp.int32).reshape(NW, NG, G, K)
    a2 = jnp.asarray(a, jnp.float32).reshape(1, 1)

    degp, = _sc_deg(dstG)
    acc1, = _sc_agg(x, srcG, dstG)
    h1 = _tc_mid(acc1, degp, x, Wl1, bl1, Wr1, a2)
    acc2, = _sc_agg(h1, srcG, dstG)
    h2 = _tc_mid(acc2, degp, h1, Wl2, bl2, Wr2, a2)
    acc3, = _sc_agg(h2, srcG, dstG)
    out = _tc_last(acc3, degp, h2, Wl3, bl3, Wr3, a2, Wp, bp)
    return out[:, 0]


# speedup vs baseline: 9.6559x; 1.0465x over previous
import functools

import jax
import jax.numpy as jnp
from jax import lax
from jax.experimental import pallas as pl
from jax.experimental.pallas import tpu as pltpu
from jax.experimental.pallas import tpu_sc as plsc

N_NODES = 10000
N_EDGES = 320000
D = 128

NC = 2
NS = 16
NW = NC * NS
E_PER_W = N_EDGES // NW
K = 50
G = 8
NG = E_PER_W // (G * K)
N_PAD = 10240
ROWS_PER_TILE = N_PAD // NS
ZROWS = 64
DEGW = 128


def _sc_agg_body(x_hbm, srcG_hbm, dstG_hbm, z_hbm, acc_out,
                 srcT, dstT, acc_sh,
                 rows0, rows1, rows2, rows3,
                 g0, g1, g2, g3, s0, s1, s2, s3, tsem):
    rows = (rows0, rows1, rows2, rows3)
    gsem = (g0, g1, g2, g3)
    ssem = (s0, s1, s2, s3)

    cid = lax.axis_index("c")
    sid = lax.axis_index("s")
    wid = sid * NC + cid

    r0 = sid * ROWS_PER_TILE
    pltpu.sync_copy(z_hbm, acc_sh.at[pl.ds(r0, ROWS_PER_TILE)])

    plsc.subcore_barrier()

    pltpu.sync_copy(srcG_hbm.at[wid, 0], srcT.at[0])
    pltpu.sync_copy(dstG_hbm.at[wid, 0], dstT.at[0])
    pltpu.async_copy(srcG_hbm.at[wid, 1], srcT.at[1], tsem)
    pltpu.async_copy(dstG_hbm.at[wid, 1], dstT.at[1], tsem)
    pltpu.async_copy(x_hbm.at[srcT.at[0, 0]], rows[0], gsem[0])
    pltpu.async_copy(x_hbm.at[srcT.at[0, 1]], rows[1], gsem[1])

    def outer(grp, _):
        h = lax.rem(grp, 2)
        for j in range(G):
            b = j % 4
            c = grp * G + j
            pltpu.make_async_copy(
                x_hbm.at[srcT.at[h, j]], rows[b], gsem[b]).wait()
            pltpu.async_copy(rows[b], acc_sh.at[dstT.at[h, j]], ssem[b],
                             add=True)
            if j == 2:
                @pl.when(jnp.logical_and(grp >= 1, grp + 1 < NG))
                def _():
                    pltpu.async_copy(srcG_hbm.at[wid, grp + 1],
                                     srcT.at[1 - h], tsem)
                    pltpu.async_copy(dstG_hbm.at[wid, grp + 1],
                                     dstT.at[1 - h], tsem)
            if j == 6:
                @pl.when(grp + 1 < NG)
                def _():
                    pltpu.make_async_copy(
                        srcG_hbm.at[wid, 0], srcT.at[0], tsem).wait()
                    pltpu.make_async_copy(
                        dstG_hbm.at[wid, 0], dstT.at[0], tsem).wait()
            b2 = (j + 2) % 4

            @pl.when(c >= 2)
            def _():
                pltpu.make_async_copy(
                    rows[b2], acc_sh.at[dstT.at[h, j]], ssem[b2]).wait()

            @pl.when(c + 2 < NG * G)
            def _():
                if j < G - 2:
                    pltpu.async_copy(x_hbm.at[srcT.at[h, j + 2]], rows[b2],
                                     gsem[b2])
                else:
                    pltpu.async_copy(x_hbm.at[srcT.at[1 - h, j - (G - 2)]],
                                     rows[b2], gsem[b2])
        return 0
    lax.fori_loop(0, NG, outer, 0)

    tot = NG * G
    for c in (tot - 2, tot - 1):
        b = c % 4
        pltpu.make_async_copy(rows[b], acc_sh.at[dstT.at[0, 0]],
                              ssem[b]).wait()

    plsc.subcore_barrier()

    pltpu.sync_copy(acc_sh.at[pl.ds(r0, ROWS_PER_TILE)],
                    acc_out.at[cid, pl.ds(r0, ROWS_PER_TILE)])


def _make_sc_agg():
    mesh = plsc.VectorSubcoreMesh(core_axis_name="c", subcore_axis_name="s",
                                  num_cores=NC, num_subcores=NS)
    return pl.kernel(
        _sc_agg_body,
        out_type=[jax.ShapeDtypeStruct((NC, N_PAD, D), jnp.float32)],
        mesh=mesh,
        scratch_types=(
            [
                pltpu.VMEM((2, G, K), jnp.int32),
                pltpu.VMEM((2, G, K), jnp.int32),
                pltpu.VMEM_SHARED((N_PAD, D), jnp.float32),
            ]
            + [pltpu.VMEM((K, D), jnp.float32) for _ in range(4)]
            + [pltpu.SemaphoreType.DMA for _ in range(9)]
        ),
    )


def _sc_deg_body(dstG_hbm, z_hbm, deg_out, dstT, onesv, deg_sh, sem):
    cid = lax.axis_index("c")
    sid = lax.axis_index("s")
    wid = sid * NC + cid

    one16 = jnp.ones((16,), jnp.float32)

    def orow(i, _):
        for j in range(DEGW // 16):
            onesv[i, pl.ds(j * 16, 16)] = one16
        return 0
    lax.fori_loop(0, K, orow, 0)
    r0 = sid * ROWS_PER_TILE
    pltpu.sync_copy(z_hbm, deg_sh.at[pl.ds(r0, ROWS_PER_TILE)])

    plsc.subcore_barrier()

    ones_rows = onesv

    def grp_body(grp, _):
        pltpu.sync_copy(dstG_hbm.at[wid, grp], dstT)
        for j in range(G):
            pltpu.async_copy(ones_rows, deg_sh.at[dstT.at[j]], sem, add=True)
        for j in range(G):
            pltpu.make_async_copy(ones_rows, deg_sh.at[dstT.at[0]],
                                  sem).wait()
        return 0
    lax.fori_loop(0, NG, grp_body, 0)

    plsc.subcore_barrier()

    pltpu.sync_copy(deg_sh.at[pl.ds(r0, ROWS_PER_TILE)],
                    deg_out.at[cid, pl.ds(r0, ROWS_PER_TILE)])


def _make_sc_deg():
    mesh = plsc.VectorSubcoreMesh(core_axis_name="c", subcore_axis_name="s",
                                  num_cores=NC, num_subcores=NS)
    return pl.kernel(
        _sc_deg_body,
        out_type=[jax.ShapeDtypeStruct((NC, N_PAD, DEGW), jnp.float32)],
        mesh=mesh,
        scratch_types=[
            pltpu.VMEM((G, K), jnp.int32),
            pltpu.VMEM((K, DEGW), jnp.float32),
            pltpu.VMEM_SHARED((N_PAD, DEGW), jnp.float32),
            pltpu.SemaphoreType.DMA,
        ],
    )


_sc_agg = _make_sc_agg()
_sc_deg = _make_sc_deg()


R_BLK = 2000


def _tc_dense_body(prelu, final, *refs):
    if final:
        (acc_ref, deg_ref, h_ref, wl_ref, bl_ref, wr_ref, a_ref,
         wp_ref, bp_ref, out_ref) = refs
    else:
        (acc_ref, deg_ref, h_ref, wl_ref, bl_ref, wr_ref, a_ref,
         out_ref) = refs
    acc = acc_ref[0] + acc_ref[1]
    deg = deg_ref[0, :, 0] + deg_ref[1, :, 0]
    mean = acc * (1.0 / jnp.clip(deg, 1.0, None))[:, None]
    h = h_ref[...]
    out = (jnp.dot(mean, wl_ref[...], preferred_element_type=jnp.float32)
           + bl_ref[...][None, :]
           + jnp.dot(h, wr_ref[...], preferred_element_type=jnp.float32))
    if prelu:
        a = a_ref[0, 0]
        out = jnp.where(out >= 0, out, a * out)
    if final:
        lvl = jnp.dot(out, wp_ref[...], preferred_element_type=jnp.float32)
        out_ref[...] = lvl + bp_ref[...][None, :]
    else:
        out_ref[...] = out


def _make_tc_dense(prelu, final):
    n_blk = N_NODES // R_BLK
    full = lambda i: (0, 0)
    in_specs = [
        pl.BlockSpec((NC, R_BLK, D), lambda i: (0, i, 0)),
        pl.BlockSpec((NC, R_BLK, DEGW), lambda i: (0, i, 0)),
        pl.BlockSpec((R_BLK, D), lambda i: (i, 0)),
        pl.BlockSpec((D, D), full),
        pl.BlockSpec((D,), lambda i: (0,)),
        pl.BlockSpec((D, D), full),
        pl.BlockSpec((1, 1), full),
    ]
    if final:
        in_specs += [
            pl.BlockSpec((D, 1), full),
            pl.BlockSpec((1,), lambda i: (0,)),
        ]
        out_spec = pl.BlockSpec((R_BLK, 1), lambda i: (i, 0))
        out_shape = jax.ShapeDtypeStruct((N_NODES, 1), jnp.float32)
    else:
        out_spec = pl.BlockSpec((R_BLK, D), lambda i: (i, 0))
        out_shape = jax.ShapeDtypeStruct((N_NODES, D), jnp.float32)
    return pl.pallas_call(
        functools.partial(_tc_dense_body, prelu, final),
        grid=(n_blk,),
        in_specs=in_specs,
        out_specs=out_spec,
        out_shape=out_shape,
    )


_tc_mid = _make_tc_dense(True, False)
_tc_last = _make_tc_dense(False, True)


def kernel(x, edge_index, Wl1, bl1, Wr1, Wl2, bl2, Wr2, Wl3, bl3, Wr3,
           a, Wp, bp):
    srcG = edge_index[0].astype(jnp.int32).reshape(NW, NG, G, K)
    dstG = edge_index[1].astype(jnp.int32).reshape(NW, NG, G, K)
    a2 = jnp.asarray(a, jnp.float32).reshape(1, 1)

    zrows = jnp.zeros((ROWS_PER_TILE, D), jnp.float32)

    degp, = _sc_deg(dstG, zrows)
    acc1, = _sc_agg(x, srcG, dstG, zrows)
    h1 = _tc_mid(acc1, degp, x, Wl1, bl1, Wr1, a2)
    acc2, = _sc_agg(h1, srcG, dstG, zrows)
    h2 = _tc_mid(acc2, degp, h1, Wl2, bl2, Wr2, a2)
    acc3, = _sc_agg(h2, srcG, dstG, zrows)
    out = _tc_last(acc3, degp, h2, Wl3, bl3, Wr3, a2, Wp, bp)
    return out[:, 0]
